# baseline (speedup 1.0000x reference)
import functools

import jax
import jax.numpy as jnp
from jax import lax
from jax.experimental import pallas as pl
from jax.experimental.pallas import tpu as pltpu

_C = 80
_ANCHORS = ((0.02, 0.03), (0.04, 0.07), (0.08, 0.06), (0.07, 0.15),
            (0.15, 0.11), (0.14, 0.29), (0.28, 0.22), (0.38, 0.48),
            (0.90, 0.78))
_L_COORD = 0.05
_L_CONF = 1.0
_L_CLS = 0.5
_GRIDS = (64, 32, 16)
_NAPS = 3
_B = 16
_N = 20
_ROWS_IM = (64 * 64 * 3, 32 * 32 * 3, 16 * 16 * 3)
_TOTAL_ROWS = _B * sum(_ROWS_IM)


def _softplus(x):
    return jnp.maximum(x, 0.0) + jnp.log1p(jnp.exp(-jnp.abs(x)))


def _box_kernel(bb_ref, sidx_ref, ci_ref, cj_ref, ai_ref, live_ref,
                tx_ref, ty_ref, tw_ref, th_ref, nobj_ref):
    bb = bb_ref[...]
    x = bb[:, :, 0]
    y = bb[:, :, 1]
    w = bb[:, :, 2]
    h = bb[:, :, 3]
    cx = x + w * 0.5
    cy = y + h * 0.5

    best = jnp.full_like(w, 1e9)
    pidx = jnp.zeros(w.shape, jnp.int32)
    for k, (aw, ah) in enumerate(_ANCHORS):
        d = jnp.abs(w - aw) + jnp.abs(h - ah)
        upd = d < best
        best = jnp.where(upd, d, best)
        pidx = jnp.where(upd, k, pidx)

    sidx = pidx // _NAPS
    aidx = pidx - sidx * _NAPS
    g = jnp.where(sidx == 0, float(_GRIDS[0]),
                  jnp.where(sidx == 1, float(_GRIDS[1]), float(_GRIDS[2])))
    ci = (cy * g).astype(jnp.int32)
    cj = (cx * g).astype(jnp.int32)
    eps = 1e-8
    ux = (cx * g - cj.astype(jnp.float32)) + eps
    uy = (cy * g - ci.astype(jnp.float32)) + eps
    tx = -jnp.log(1.0 / ux - 1.0)
    ty = -jnp.log(1.0 / uy - 1.0)

    aw_sel = jnp.full_like(w, _ANCHORS[0][0])
    ah_sel = jnp.full_like(w, _ANCHORS[0][1])
    for k, (aw, ah) in enumerate(_ANCHORS):
        if k == 0:
            continue
        aw_sel = jnp.where(pidx == k, aw, aw_sel)
        ah_sel = jnp.where(pidx == k, ah, ah_sel)
    tw = jnp.log(w / aw_sel)
    th = jnp.log(h / ah_sel)

    gi = g.astype(jnp.int32)
    rowid = (ci * gi + cj) * _NAPS + aidx
    offs = jnp.where(sidx == 0, 0,
                     jnp.where(sidx == 1, _ROWS_IM[0],
                               _ROWS_IM[0] + _ROWS_IM[1]))
    key = rowid + offs

    ka = key[:, :, None]
    kb = key[:, None, :]
    jj = lax.broadcasted_iota(jnp.int32, (_B, _N, _N), 1)
    jp = lax.broadcasted_iota(jnp.int32, (_B, _N, _N), 2)
    clobbered = jnp.any((ka == kb) & (jp > jj), axis=2)
    live = jnp.where(clobbered, 0, 1)

    sidx_ref[...] = sidx
    ci_ref[...] = ci
    cj_ref[...] = cj
    ai_ref[...] = aidx
    live_ref[...] = live
    tx_ref[...] = tx[:, None, :]
    ty_ref[...] = ty[:, None, :]
    tw_ref[...] = tw[:, None, :]
    th_ref[...] = th[:, None, :]
    nobj_ref[...] = jnp.sum(live.astype(jnp.float32)).reshape(1, 1)


def _sweep5_kernel(pred_ref, sidx_ref, ci_ref, cj_ref, ai_ref, live_ref,
                   conf_ref, cand_ref, *, s, bh, W):
    i = pl.program_id(0)
    hb = pl.program_id(1)
    blk = pred_ref[0]

    x4 = blk[:, :, :, 4:5]
    local = jnp.sum(_softplus(x4))

    @pl.when((i == 0) & (hb == 0))
    def _():
        conf_ref[...] = jnp.zeros_like(conf_ref)

    conf_ref[...] += local.reshape(1, 1)

    @pl.when(hb == 0)
    def _():
        cand_ref[...] = jnp.zeros_like(cand_ref)

    a3 = lax.broadcasted_iota(jnp.int32, (_NAPS, 1), 0)
    for j in range(_N):
        cij = ci_ref[i, j]
        cjj = cj_ref[i, j]
        aj = ai_ref[i, j]
        ok = ((sidx_ref[i, j] == s) & (live_ref[i, j] == 1)
              & (cij >= hb * bh) & (cij < (hb + 1) * bh))

        @pl.when(ok)
        def _():
            rows3 = pred_ref[0, pl.ds(cij - hb * bh, 1),
                             pl.ds(cjj, 1)]
            rows3 = rows3[0, 0]
            sel = (a3 == aj).astype(jnp.float32)
            row = jnp.sum(rows3 * sel, axis=0, keepdims=True)
            cand_ref[0, j:j + 1, :] = row


def _fin_kernel(c0_ref, c1_ref, c2_ref, tx_ref, ty_ref, tw_ref, th_ref,
                live_ref, lab_ref, nobj_ref, s0_ref, s1_ref, s2_ref,
                loss_ref, coord_ref, conf_ref, cls_ref):
    p = c0_ref[...] + c1_ref[...] + c2_ref[...]
    lv = live_ref[...].astype(jnp.float32)

    d0 = p[:, :, 0] - tx_ref[...][:, 0, :]
    d1 = p[:, :, 1] - ty_ref[...][:, 0, :]
    d2 = p[:, :, 2] - tw_ref[...][:, 0, :]
    d3 = p[:, :, 3] - th_ref[...][:, 0, :]
    coord_sum = jnp.sum(lv * (d0 * d0 + d1 * d1 + d2 * d2 + d3 * d3))

    conf_corr = -jnp.sum(lv * p[:, :, 4])

    lab = lab_ref[...][:, 0, :]
    oh = (lax.broadcasted_iota(jnp.int32, (_B, _N, _C), 2)
          == lab[:, :, None]).astype(jnp.float32)
    pc = p[:, :, 5:]
    bce = jnp.maximum(pc, 0.0) - pc * oh + jnp.log1p(jnp.exp(-jnp.abs(pc)))
    class_sum = jnp.sum(bce * lv[:, :, None])

    nobj = nobj_ref[...]
    conf_total = s0_ref[...] + s1_ref[...] + s2_ref[...] \
        + conf_corr.reshape(1, 1)
    conf_loss = _L_CONF * conf_total / float(_TOTAL_ROWS)
    coord_loss = _L_COORD * coord_sum.reshape(1, 1) / (nobj * 4.0)
    class_loss = _L_CLS * class_sum.reshape(1, 1) / (nobj * float(_C))

    coord_ref[...] = coord_loss
    conf_ref[...] = conf_loss
    cls_ref[...] = class_loss
    loss_ref[...] = coord_loss + conf_loss + class_loss


def _run_box(bboxes):
    f3 = jax.ShapeDtypeStruct((_B, 1, _N), jnp.float32)
    i2 = jax.ShapeDtypeStruct((_B, _N), jnp.int32)
    return pl.pallas_call(
        _box_kernel,
        out_shape=(i2, i2, i2, i2, i2, f3, f3, f3, f3,
                   jax.ShapeDtypeStruct((1, 1), jnp.float32)),
    )(bboxes)


def _run_sweep5(pred, sidx, ci, cj, ai, live, s, bh):
    H = _GRIDS[s]
    W = _GRIDS[s]
    nhb = H // bh
    smem = pl.BlockSpec(memory_space=pltpu.SMEM)
    conf, cand = pl.pallas_call(
        functools.partial(_sweep5_kernel, s=s, bh=bh, W=W),
        grid=(_B, nhb),
        in_specs=[
            pl.BlockSpec((1, bh, W, _NAPS, 5 + _C),
                         lambda i, hb: (i, hb, 0, 0, 0)),
            smem, smem, smem, smem, smem,
        ],
        out_specs=(
            pl.BlockSpec((1, 1), lambda i, hb: (0, 0)),
            pl.BlockSpec((1, _N, 5 + _C), lambda i, hb: (i, 0, 0)),
        ),
        out_shape=(
            jax.ShapeDtypeStruct((1, 1), jnp.float32),
            jax.ShapeDtypeStruct((_B, _N, 5 + _C), jnp.float32),
        ),
    )(pred, sidx, ci, cj, ai, live)
    return conf, cand


def kernel(pred_s0, pred_s1, pred_s2, bboxes, labels):
    preds = (pred_s0, pred_s1, pred_s2)
    sidx, ci, cj, ai, live, tx, ty, tw, th, nobj = _run_box(bboxes)

    bhs = (16, 32, 16)
    confs = []
    cands = []
    for s in range(3):
        conf, cand = _run_sweep5(preds[s], sidx, ci, cj, ai, live, s, bhs[s])
        confs.append(conf)
        cands.append(cand)

    lab3 = labels.astype(jnp.int32).reshape(_B, 1, _N)
    sc = jax.ShapeDtypeStruct((1, 1), jnp.float32)
    loss, coord, conf, cls = pl.pallas_call(
        _fin_kernel,
        out_shape=(sc, sc, sc, sc),
    )(cands[0], cands[1], cands[2], tx, ty, tw, th, live, lab3, nobj,
      confs[0], confs[1], confs[2])

    return (loss[0, 0], coord[0, 0], conf[0, 0], cls[0, 0])


# speedup vs baseline: 1.1773x; 1.1773x over previous
import functools

import jax
import jax.numpy as jnp
from jax import lax
from jax.experimental import pallas as pl

_C = 80
_D = 5 + _C
_ANCHORS = ((0.02, 0.03), (0.04, 0.07), (0.08, 0.06), (0.07, 0.15),
            (0.15, 0.11), (0.14, 0.29), (0.28, 0.22), (0.38, 0.48),
            (0.90, 0.78))
_L_COORD = 0.05
_L_CONF = 1.0
_L_CLS = 0.5
_GRIDS = (64, 32, 16)
_NAPS = 3
_B = 16
_N = 20
_ROWS_IM = (64 * 64 * 3, 32 * 32 * 3, 16 * 16 * 3)
_TOTAL_ROWS = _B * sum(_ROWS_IM)


def _softplus(x):
    return jnp.maximum(x, 0.0) + jnp.log1p(jnp.exp(-jnp.abs(x)))


def _box_kernel(bb_ref, sidx_ref, rowid_ref, live_ref,
                tx_ref, ty_ref, tw_ref, th_ref, nobj_ref):
    bb = bb_ref[...]
    x = bb[:, :, 0]
    y = bb[:, :, 1]
    w = bb[:, :, 2]
    h = bb[:, :, 3]
    cx = x + w * 0.5
    cy = y + h * 0.5

    best = jnp.full_like(w, 1e9)
    pidx = jnp.zeros(w.shape, jnp.int32)
    for k, (aw, ah) in enumerate(_ANCHORS):
        d = jnp.abs(w - aw) + jnp.abs(h - ah)
        upd = d < best
        best = jnp.where(upd, d, best)
        pidx = jnp.where(upd, k, pidx)

    sidx = pidx // _NAPS
    aidx = pidx - sidx * _NAPS
    g = jnp.where(sidx == 0, float(_GRIDS[0]),
                  jnp.where(sidx == 1, float(_GRIDS[1]), float(_GRIDS[2])))
    ci = (cy * g).astype(jnp.int32)
    cj = (cx * g).astype(jnp.int32)
    eps = 1e-8
    ux = (cx * g - cj.astype(jnp.float32)) + eps
    uy = (cy * g - ci.astype(jnp.float32)) + eps
    tx = -jnp.log(1.0 / ux - 1.0)
    ty = -jnp.log(1.0 / uy - 1.0)

    aw_sel = jnp.full_like(w, _ANCHORS[0][0])
    ah_sel = jnp.full_like(w, _ANCHORS[0][1])
    for k, (aw, ah) in enumerate(_ANCHORS):
        if k == 0:
            continue
        aw_sel = jnp.where(pidx == k, aw, aw_sel)
        ah_sel = jnp.where(pidx == k, ah, ah_sel)
    tw = jnp.log(w / aw_sel)
    th = jnp.log(h / ah_sel)

    gi = g.astype(jnp.int32)
    rowid = (ci * gi + cj) * _NAPS + aidx
    offs = jnp.where(sidx == 0, 0,
                     jnp.where(sidx == 1, _ROWS_IM[0],
                               _ROWS_IM[0] + _ROWS_IM[1]))
    key = rowid + offs

    ka = key[:, :, None]
    kb = key[:, None, :]
    jj = lax.broadcasted_iota(jnp.int32, (_B, _N, _N), 1)
    jp = lax.broadcasted_iota(jnp.int32, (_B, _N, _N), 2)
    clobbered = jnp.any((ka == kb) & (jp > jj), axis=2)
    live = jnp.where(clobbered, 0.0, 1.0)

    sidx_ref[...] = sidx[:, None, :]
    rowid_ref[...] = rowid[:, None, :]
    live_ref[...] = live[:, None, :]
    tx_ref[...] = tx[:, None, :]
    ty_ref[...] = ty[:, None, :]
    tw_ref[...] = tw[:, None, :]
    th_ref[...] = th[:, None, :]
    nobj_ref[...] = jnp.sum(live).reshape(1, 1)


def _sweep_kernel(pred_ref, sidx_ref, rowid_ref, live_ref,
                  conf_ref, cand_ref, *, s, bq):
    i = pl.program_id(0)
    hb = pl.program_id(1)
    blk = pred_ref[0]

    x4 = jnp.concatenate(
        [blk[:, 4:5], blk[:, 89:90], blk[:, 174:175]], axis=1)
    local = jnp.sum(_softplus(x4))

    @pl.when((i == 0) & (hb == 0))
    def _():
        conf_ref[...] = jnp.zeros_like(conf_ref)

    conf_ref[...] += local.reshape(1, 1)

    rid = rowid_ref[0, 0, :]
    sid = sidx_ref[0, 0, :]
    lv = live_ref[0, 0, :]
    q = rid // 3
    k3 = rid - q * 3
    qiota = lax.broadcasted_iota(jnp.int32, (_N, bq), 1) + hb * bq
    m = (q[:, None] == qiota) & (sid[:, None] == s) & (lv[:, None] > 0.0)
    mf = m.astype(jnp.float32)
    g3 = lax.dot_general(mf, blk, (((1,), (0,)), ((), ())),
                         preferred_element_type=jnp.float32)
    kc = k3[:, None]
    g = jnp.where(kc == 0, g3[:, 0:_D],
                  jnp.where(kc == 1, g3[:, _D:2 * _D], g3[:, 2 * _D:3 * _D]))

    @pl.when(hb == 0)
    def _():
        cand_ref[...] = jnp.zeros_like(cand_ref)

    cand_ref[...] += g[None]


def _fin_kernel(c0_ref, c1_ref, c2_ref, tx_ref, ty_ref, tw_ref, th_ref,
                live_ref, lab_ref, nobj_ref, s0_ref, s1_ref, s2_ref,
                loss_ref, coord_ref, conf_ref, cls_ref):
    p = c0_ref[...] + c1_ref[...] + c2_ref[...]
    lv = live_ref[...][:, 0, :]

    d0 = p[:, :, 0] - tx_ref[...][:, 0, :]
    d1 = p[:, :, 1] - ty_ref[...][:, 0, :]
    d2 = p[:, :, 2] - tw_ref[...][:, 0, :]
    d3 = p[:, :, 3] - th_ref[...][:, 0, :]
    coord_sum = jnp.sum(lv * (d0 * d0 + d1 * d1 + d2 * d2 + d3 * d3))

    conf_corr = -jnp.sum(lv * p[:, :, 4])

    lab = lab_ref[...][:, 0, :]
    oh = (lax.broadcasted_iota(jnp.int32, (_B, _N, _C), 2)
          == lab[:, :, None]).astype(jnp.float32)
    pc = p[:, :, 5:]
    bce = jnp.maximum(pc, 0.0) - pc * oh + jnp.log1p(jnp.exp(-jnp.abs(pc)))
    class_sum = jnp.sum(bce * lv[:, :, None])

    nobj = nobj_ref[...]
    conf_total = s0_ref[...] + s1_ref[...] + s2_ref[...] \
        + conf_corr.reshape(1, 1)
    conf_loss = _L_CONF * conf_total / float(_TOTAL_ROWS)
    coord_loss = _L_COORD * coord_sum.reshape(1, 1) / (nobj * 4.0)
    class_loss = _L_CLS * class_sum.reshape(1, 1) / (nobj * float(_C))

    coord_ref[...] = coord_loss
    conf_ref[...] = conf_loss
    cls_ref[...] = class_loss
    loss_ref[...] = coord_loss + conf_loss + class_loss


def _run_box(bboxes):
    f = jax.ShapeDtypeStruct((_B, 1, _N), jnp.float32)
    ii = jax.ShapeDtypeStruct((_B, 1, _N), jnp.int32)
    return pl.pallas_call(
        _box_kernel,
        out_shape=(ii, ii, f, f, f, f, f,
                   jax.ShapeDtypeStruct((1, 1), jnp.float32)),
    )(bboxes)


def _run_sweep(pred, sidx, rowid, live, s, bq):
    nq = _ROWS_IM[s] // 3
    nblk = nq // bq
    small = pl.BlockSpec((1, 1, _N), lambda i, hb: (i, 0, 0))
    conf, cand = pl.pallas_call(
        functools.partial(_sweep_kernel, s=s, bq=bq),
        grid=(_B, nblk),
        in_specs=[
            pl.BlockSpec((1, bq, 3 * _D), lambda i, hb: (i, hb, 0)),
            small, small, small,
        ],
        out_specs=(
            pl.BlockSpec((1, 1), lambda i, hb: (0, 0)),
            pl.BlockSpec((1, _N, _D), lambda i, hb: (i, 0, 0)),
        ),
        out_shape=(
            jax.ShapeDtypeStruct((1, 1), jnp.float32),
            jax.ShapeDtypeStruct((_B, _N, _D), jnp.float32),
        ),
    )(pred, sidx, rowid, live)
    return conf, cand


def kernel(pred_s0, pred_s1, pred_s2, bboxes, labels):
    preds = [
        pred_s0.reshape(_B, _ROWS_IM[0] // 3, 3 * _D),
        pred_s1.reshape(_B, _ROWS_IM[1] // 3, 3 * _D),
        pred_s2.reshape(_B, _ROWS_IM[2] // 3, 3 * _D),
    ]
    sidx, rowid, live, tx, ty, tw, th, nobj = _run_box(bboxes)

    bqs = (2048, 1024, 256)
    confs = []
    cands = []
    for s in range(3):
        conf, cand = _run_sweep(preds[s], sidx, rowid, live, s, bqs[s])
        confs.append(conf)
        cands.append(cand)

    lab3 = labels.astype(jnp.int32).reshape(_B, 1, _N)
    sc = jax.ShapeDtypeStruct((1, 1), jnp.float32)
    loss, coord, conf, cls = pl.pallas_call(
        _fin_kernel,
        out_shape=(sc, sc, sc, sc),
    )(cands[0], cands[1], cands[2], tx, ty, tw, th, live, lab3, nobj,
      confs[0], confs[1], confs[2])

    return (loss[0, 0], coord[0, 0], conf[0, 0], cls[0, 0])


# speedup vs baseline: 1.4612x; 1.2411x over previous
import functools

import jax
import jax.numpy as jnp
from jax import lax
from jax.experimental import pallas as pl

_C = 80
_ANCHORS = ((0.02, 0.03), (0.04, 0.07), (0.08, 0.06), (0.07, 0.15),
            (0.15, 0.11), (0.14, 0.29), (0.28, 0.22), (0.38, 0.48),
            (0.90, 0.78))
_L_COORD = 0.05
_L_CONF = 1.0
_L_CLS = 0.5
_GRIDS = (64, 32, 16)
_NAPS = 3
_B = 16
_N = 20
_ROWS_IM = (64 * 64 * 3, 32 * 32 * 3, 16 * 16 * 3)
_TOTAL_ROWS = _B * sum(_ROWS_IM)


def _softplus(x):
    return jnp.maximum(x, 0.0) + jnp.log1p(jnp.exp(-jnp.abs(x)))


def _box_kernel(bb_ref, sidx_ref, rowid_ref, live_ref,
                tx_ref, ty_ref, tw_ref, th_ref, nobj_ref):
    bb = bb_ref[...]
    x = bb[:, :, 0]
    y = bb[:, :, 1]
    w = bb[:, :, 2]
    h = bb[:, :, 3]
    cx = x + w * 0.5
    cy = y + h * 0.5

    best = jnp.full_like(w, 1e9)
    pidx = jnp.zeros(w.shape, jnp.int32)
    for k, (aw, ah) in enumerate(_ANCHORS):
        d = jnp.abs(w - aw) + jnp.abs(h - ah)
        upd = d < best
        best = jnp.where(upd, d, best)
        pidx = jnp.where(upd, k, pidx)

    sidx = pidx // _NAPS
    aidx = pidx - sidx * _NAPS
    g = jnp.where(sidx == 0, float(_GRIDS[0]),
                  jnp.where(sidx == 1, float(_GRIDS[1]), float(_GRIDS[2])))
    ci = (cy * g).astype(jnp.int32)
    cj = (cx * g).astype(jnp.int32)
    eps = 1e-8
    ux = (cx * g - cj.astype(jnp.float32)) + eps
    uy = (cy * g - ci.astype(jnp.float32)) + eps
    tx = -jnp.log(1.0 / ux - 1.0)
    ty = -jnp.log(1.0 / uy - 1.0)

    aw_sel = jnp.full_like(w, _ANCHORS[0][0])
    ah_sel = jnp.full_like(w, _ANCHORS[0][1])
    for k, (aw, ah) in enumerate(_ANCHORS):
        if k == 0:
            continue
        aw_sel = jnp.where(pidx == k, aw, aw_sel)
        ah_sel = jnp.where(pidx == k, ah, ah_sel)
    tw = jnp.log(w / aw_sel)
    th = jnp.log(h / ah_sel)

    gi = g.astype(jnp.int32)
    rowid = (ci * gi + cj) * _NAPS + aidx
    offs = jnp.where(sidx == 0, 0,
                     jnp.where(sidx == 1, _ROWS_IM[0],
                               _ROWS_IM[0] + _ROWS_IM[1]))
    key = rowid + offs

    ka = key[:, :, None]
    kb = key[:, None, :]
    jj = lax.broadcasted_iota(jnp.int32, (_B, _N, _N), 1)
    jp = lax.broadcasted_iota(jnp.int32, (_B, _N, _N), 2)
    clobbered = jnp.any((ka == kb) & (jp > jj), axis=2)
    live = jnp.where(clobbered, 0.0, 1.0)

    sidx_ref[...] = sidx[:, None, :]
    rowid_ref[...] = rowid[:, None, :]
    live_ref[...] = live[:, None, :]
    tx_ref[...] = tx[:, None, :]
    ty_ref[...] = ty[:, None, :]
    tw_ref[...] = tw[:, None, :]
    th_ref[...] = th[:, None, :]
    nobj_ref[...] = jnp.sum(live).reshape(1, 1)


def _sweep_kernel(pred_ref, sidx_ref, rowid_ref, live_ref,
                  conf_ref, cand_ref, *, s, bs):
    i = pl.program_id(0)
    hb = pl.program_id(1)
    blk = pred_ref[0]

    x4 = blk[:, 4].reshape(bs // 128, 128)
    local = jnp.sum(_softplus(x4))

    @pl.when((i == 0) & (hb == 0))
    def _():
        conf_ref[...] = jnp.zeros_like(conf_ref)

    conf_ref[...] += local.reshape(1, 1)

    rid = rowid_ref[0, 0, :]
    sid = sidx_ref[0, 0, :]
    lv = live_ref[0, 0, :]
    riota = lax.broadcasted_iota(jnp.int32, (_N, bs), 1) + hb * bs
    m = (rid[:, None] == riota) & (sid[:, None] == s) & (lv[:, None] > 0.0)
    mf = m.astype(jnp.float32)
    g = lax.dot_general(mf, blk, (((1,), (0,)), ((), ())),
                        preferred_element_type=jnp.float32)

    @pl.when(hb == 0)
    def _():
        cand_ref[...] = jnp.zeros_like(cand_ref)

    cand_ref[...] += g[None]


def _fin_kernel(c0_ref, c1_ref, c2_ref, tx_ref, ty_ref, tw_ref, th_ref,
                live_ref, lab_ref, nobj_ref, s0_ref, s1_ref, s2_ref,
                loss_ref, coord_ref, conf_ref, cls_ref):
    p = c0_ref[...] + c1_ref[...] + c2_ref[...]
    lv = live_ref[...][:, 0, :]

    d0 = p[:, :, 0] - tx_ref[...][:, 0, :]
    d1 = p[:, :, 1] - ty_ref[...][:, 0, :]
    d2 = p[:, :, 2] - tw_ref[...][:, 0, :]
    d3 = p[:, :, 3] - th_ref[...][:, 0, :]
    coord_sum = jnp.sum(lv * (d0 * d0 + d1 * d1 + d2 * d2 + d3 * d3))

    conf_corr = -jnp.sum(lv * p[:, :, 4])

    lab = lab_ref[...][:, 0, :]
    oh = (lax.broadcasted_iota(jnp.int32, (_B, _N, _C), 2)
          == lab[:, :, None]).astype(jnp.float32)
    pc = p[:, :, 5:]
    bce = jnp.maximum(pc, 0.0) - pc * oh + jnp.log1p(jnp.exp(-jnp.abs(pc)))
    class_sum = jnp.sum(bce * lv[:, :, None])

    nobj = nobj_ref[...]
    conf_total = s0_ref[...] + s1_ref[...] + s2_ref[...] \
        + conf_corr.reshape(1, 1)
    conf_loss = _L_CONF * conf_total / float(_TOTAL_ROWS)
    coord_loss = _L_COORD * coord_sum.reshape(1, 1) / (nobj * 4.0)
    class_loss = _L_CLS * class_sum.reshape(1, 1) / (nobj * float(_C))

    coord_ref[...] = coord_loss
    conf_ref[...] = conf_loss
    cls_ref[...] = class_loss
    loss_ref[...] = coord_loss + conf_loss + class_loss


def _run_box(bboxes):
    f = jax.ShapeDtypeStruct((_B, 1, _N), jnp.float32)
    ii = jax.ShapeDtypeStruct((_B, 1, _N), jnp.int32)
    return pl.pallas_call(
        _box_kernel,
        out_shape=(ii, ii, f, f, f, f, f,
                   jax.ShapeDtypeStruct((1, 1), jnp.float32)),
    )(bboxes)


def _run_sweep(pred, sidx, rowid, live, s, bs):
    rows_im = _ROWS_IM[s]
    nblk = rows_im // bs
    small = pl.BlockSpec((1, 1, _N), lambda i, hb: (i, 0, 0))
    conf, cand = pl.pallas_call(
        functools.partial(_sweep_kernel, s=s, bs=bs),
        grid=(_B, nblk),
        in_specs=[
            pl.BlockSpec((1, bs, 5 + _C), lambda i, hb: (i, hb, 0)),
            small, small, small,
        ],
        out_specs=(
            pl.BlockSpec((1, 1), lambda i, hb: (0, 0)),
            pl.BlockSpec((1, _N, 5 + _C), lambda i, hb: (i, 0, 0)),
        ),
        out_shape=(
            jax.ShapeDtypeStruct((1, 1), jnp.float32),
            jax.ShapeDtypeStruct((_B, _N, 5 + _C), jnp.float32),
        ),
    )(pred, sidx, rowid, live)
    return conf, cand


def kernel(pred_s0, pred_s1, pred_s2, bboxes, labels):
    preds = [
        pred_s0.reshape(_B, _ROWS_IM[0], 5 + _C),
        pred_s1.reshape(_B, _ROWS_IM[1], 5 + _C),
        pred_s2.reshape(_B, _ROWS_IM[2], 5 + _C),
    ]
    sidx, rowid, live, tx, ty, tw, th, nobj = _run_box(bboxes)

    bss = (4096, 3072, 768)
    confs = []
    cands = []
    for s in range(3):
        conf, cand = _run_sweep(preds[s], sidx, rowid, live, s, bss[s])
        confs.append(conf)
        cands.append(cand)

    lab3 = labels.astype(jnp.int32).reshape(_B, 1, _N)
    sc = jax.ShapeDtypeStruct((1, 1), jnp.float32)
    loss, coord, conf, cls = pl.pallas_call(
        _fin_kernel,
        out_shape=(sc, sc, sc, sc),
    )(cands[0], cands[1], cands[2], tx, ty, tw, th, live, lab3, nobj,
      confs[0], confs[1], confs[2])

    return (loss[0, 0], coord[0, 0], conf[0, 0], cls[0, 0])
